# baseline (speedup 1.0000x reference)
import dataclasses

import jax
import jax.numpy as jnp
from jax.experimental import pallas as pl
from jax.experimental.pallas import tpu as pltpu
from jax.experimental.pallas import tpu_sc as plsc

_NB = 8
_BM = 1024
_BN = 512

_NC = 2
_NS = 16
_L = 16


def _mlp_block(z_ref, w1_ref, b1_ref, w2_ref, b2_ref, logits_ref, acc_ref):
    j = pl.program_id(1)
    nj = pl.num_programs(1)

    h = jax.lax.dot_general(
        z_ref[...], w1_ref[...], (((1,), (1,)), ((), ())),
        preferred_element_type=jnp.float32)
    h = h + b1_ref[...]
    h = 0.5 * h * (1.0 + jax.lax.erf(h * 0.7071067811865476))
    part = jax.lax.dot_general(
        h, w2_ref[...], (((1,), (1,)), ((), ())),
        preferred_element_type=jnp.float32)

    @pl.when(j == 0)
    def _init():
        acc_ref[...] = part

    @pl.when(j > 0)
    def _accum():
        acc_ref[...] += part

    @pl.when(j == nj - 1)
    def _write():
        logits_ref[...] = acc_ref[...] + b2_ref[...]


def _tc_logits(z, W1, b1, W2, b2):
    n, d = z.shape
    b1r = jnp.reshape(b1, (1, d))
    b2r = jnp.reshape(b2, (1, _NB))
    grid = (n // _BM, d // _BN)
    return pl.pallas_call(
        _mlp_block,
        grid=grid,
        in_specs=[
            pl.BlockSpec((_BM, d), lambda i, j: (i, 0)),
            pl.BlockSpec((_BN, d), lambda i, j: (j, 0)),
            pl.BlockSpec((1, _BN), lambda i, j: (0, j)),
            pl.BlockSpec((_NB, _BN), lambda i, j: (0, j)),
            pl.BlockSpec((1, _NB), lambda i, j: (0, 0)),
        ],
        out_specs=pl.BlockSpec((_BM, _NB), lambda i, j: (i, 0)),
        out_shape=jax.ShapeDtypeStruct((n, _NB), jnp.float32),
        scratch_shapes=[pltpu.VMEM((_BM, _NB), jnp.float32)],
        compiler_params=pltpu.CompilerParams(
            dimension_semantics=("parallel", "arbitrary")),
    )(z, W1, b1r, W2, b2r)


def _sc_route(logits, inv_temp_vec):
    n = logits.shape[0]
    rpw = n // (_NC * _NS)

    def body(lg_hbm, invt_hbm, w_hbm, idx_hbm, lg_v, w_v, ix_v, invt_v):
        wid = jax.lax.axis_index("s") * _NC + jax.lax.axis_index("c")
        base = wid * rpw
        pltpu.sync_copy(invt_hbm, invt_v)
        pltpu.sync_copy(lg_hbm.at[pl.ds(base, rpw)], lg_v)
        inv_t = invt_v[...]
        iota = jax.lax.iota(jnp.int32, _L)

        @pl.loop(0, rpw, step=_L)
        def _(c):
            rows = iota + c
            cols = [jnp.full((_L,), e, jnp.int32) for e in range(_NB)]
            l = [plsc.load_gather(lg_v, [rows, cols[e]]) for e in range(_NB)]
            m1 = l[0]
            for e in range(1, _NB):
                m1 = jnp.maximum(m1, l[e])
            idx1 = jnp.full((_L,), 0, jnp.int32)
            for e in range(_NB - 1, -1, -1):
                idx1 = jnp.where(l[e] == m1, e, idx1)
            neg = jnp.full((_L,), -jnp.inf, jnp.float32)
            m2 = neg
            for e in range(_NB):
                m2 = jnp.maximum(m2, jnp.where(idx1 == e, neg, l[e]))
            idx2 = jnp.full((_L,), 0, jnp.int32)
            for e in range(_NB - 1, -1, -1):
                hit = jnp.logical_and(l[e] == m2, idx1 != e)
                idx2 = jnp.where(hit, e, idx2)
            ex = jnp.exp((m2 - m1) * inv_t)
            denom = 1.0 + ex
            w_hi = 1.0 / denom
            w_lo = ex / denom
            zero = jnp.zeros((_L,), jnp.float32)
            for e in range(_NB):
                we = jnp.where(idx1 == e, w_hi,
                               jnp.where(idx2 == e, w_lo, zero))
                plsc.store_scatter(w_v, [rows, cols[e]], we)
            plsc.store_scatter(ix_v, [rows, cols[0]], idx1)
            plsc.store_scatter(ix_v, [rows, cols[1]], idx2)

        pltpu.sync_copy(w_v, w_hbm.at[pl.ds(base, rpw)])
        pltpu.sync_copy(ix_v, idx_hbm.at[pl.ds(base, rpw)])

    cp = pltpu.CompilerParams()
    fields = pltpu.CompilerParams.__dataclass_fields__
    if "needs_layout_passes" in fields:
        cp = dataclasses.replace(cp, needs_layout_passes=False)
    if "use_tc_tiling_on_sc" in fields:
        cp = dataclasses.replace(cp, use_tc_tiling_on_sc=False)
    route = pl.kernel(
        body,
        out_type=[
            jax.ShapeDtypeStruct((n, _NB), jnp.float32),
            jax.ShapeDtypeStruct((n, 2), jnp.int32),
        ],
        mesh=plsc.VectorSubcoreMesh(core_axis_name="c", subcore_axis_name="s"),
        scratch_types=[
            pltpu.VMEM((rpw, _NB), jnp.float32),
            pltpu.VMEM((rpw, _NB), jnp.float32),
            pltpu.VMEM((rpw, 2), jnp.int32),
            pltpu.VMEM((_L,), jnp.float32),
        ],
        compiler_params=cp,
    )
    return route(logits, inv_temp_vec)


@jax.jit
def kernel(z, W1, b1, W2, b2, temperature):
    inv_temp = 1.0 / (jax.nn.softplus(temperature) + 0.1)
    inv_temp_vec = jnp.full((_L,), inv_temp, jnp.float32)
    logits = _tc_logits(z, W1, b1, W2, b2)
    weights, idx = _sc_route(logits, inv_temp_vec)
    return weights, idx


# speedup vs baseline: 1.0279x; 1.0279x over previous
import dataclasses

import jax
import jax.numpy as jnp
from jax.experimental import pallas as pl
from jax.experimental.pallas import tpu as pltpu
from jax.experimental.pallas import tpu_sc as plsc

_NB = 8
_BM = 1024
_BN = 512

_NC = 2
_NS = 16
_L = 16


def _mlp_block(z_ref, w1_ref, b1_ref, w2_ref, b2_ref, logits_ref, acc_ref):
    j = pl.program_id(1)
    nj = pl.num_programs(1)

    h = jax.lax.dot_general(
        z_ref[...], w1_ref[...], (((1,), (1,)), ((), ())),
        preferred_element_type=jnp.float32)
    h = h + b1_ref[...]
    h = 0.5 * h * (1.0 + jax.lax.erf(h * 0.7071067811865476))
    part = jax.lax.dot_general(
        w2_ref[...], h, (((1,), (1,)), ((), ())),
        preferred_element_type=jnp.float32)

    @pl.when(j == 0)
    def _init():
        acc_ref[...] = part

    @pl.when(j > 0)
    def _accum():
        acc_ref[...] += part

    @pl.when(j == nj - 1)
    def _write():
        logits_ref[...] = acc_ref[...] + b2_ref[...]


def _tc_logits(z, W1, b1, W2, b2):
    n, d = z.shape
    b1r = jnp.reshape(b1, (1, d))
    b2r = jnp.reshape(b2, (_NB, 1))
    grid = (n // _BM, d // _BN)
    return pl.pallas_call(
        _mlp_block,
        grid=grid,
        in_specs=[
            pl.BlockSpec((_BM, d), lambda i, j: (i, 0)),
            pl.BlockSpec((_BN, d), lambda i, j: (j, 0)),
            pl.BlockSpec((1, _BN), lambda i, j: (0, j)),
            pl.BlockSpec((_NB, _BN), lambda i, j: (0, j)),
            pl.BlockSpec((_NB, 1), lambda i, j: (0, 0)),
        ],
        out_specs=pl.BlockSpec((_NB, _BM), lambda i, j: (0, i)),
        out_shape=jax.ShapeDtypeStruct((_NB, n), jnp.float32),
        scratch_shapes=[pltpu.VMEM((_NB, _BM), jnp.float32)],
        compiler_params=pltpu.CompilerParams(
            dimension_semantics=("parallel", "arbitrary")),
    )(z, W1, b1r, W2, b2r)


def _sc_route(logits_t, inv_temp_vec):
    n = logits_t.shape[1]
    rpw = n // (_NC * _NS)

    def body(lg_hbm, invt_hbm, w_hbm, idx_hbm, lg_v, w_v, ix_v, invt_v):
        wid = jax.lax.axis_index("s") * _NC + jax.lax.axis_index("c")
        base = wid * rpw
        pltpu.sync_copy(invt_hbm, invt_v)
        pltpu.sync_copy(lg_hbm.at[:, pl.ds(base, rpw)], lg_v)
        inv_t = invt_v[...]
        iota = jax.lax.iota(jnp.int32, _L)

        @pl.loop(0, rpw, step=_L)
        def _(c):
            rows = iota + c
            cols = [jnp.full((_L,), e, jnp.int32) for e in range(_NB)]
            l = [lg_v[e, pl.ds(c, _L)] for e in range(_NB)]
            m1 = l[0]
            for e in range(1, _NB):
                m1 = jnp.maximum(m1, l[e])
            idx1 = jnp.full((_L,), 0, jnp.int32)
            for e in range(_NB - 1, -1, -1):
                idx1 = jnp.where(l[e] == m1, e, idx1)
            neg = jnp.full((_L,), -jnp.inf, jnp.float32)
            m2 = neg
            for e in range(_NB):
                m2 = jnp.maximum(m2, jnp.where(idx1 == e, neg, l[e]))
            idx2 = jnp.full((_L,), 0, jnp.int32)
            for e in range(_NB - 1, -1, -1):
                hit = jnp.logical_and(l[e] == m2, idx1 != e)
                idx2 = jnp.where(hit, e, idx2)
            ex = jnp.exp((m2 - m1) * inv_t)
            denom = 1.0 + ex
            w_hi = 1.0 / denom
            w_lo = ex / denom
            zero = jnp.zeros((_L,), jnp.float32)
            for e in range(_NB):
                we = jnp.where(idx1 == e, w_hi,
                               jnp.where(idx2 == e, w_lo, zero))
                plsc.store_scatter(w_v, [rows, cols[e]], we)
            plsc.store_scatter(ix_v, [rows, cols[0]], idx1)
            plsc.store_scatter(ix_v, [rows, cols[1]], idx2)

        pltpu.sync_copy(w_v, w_hbm.at[pl.ds(base, rpw)])
        pltpu.sync_copy(ix_v, idx_hbm.at[pl.ds(base, rpw)])

    cp = pltpu.CompilerParams()
    fields = pltpu.CompilerParams.__dataclass_fields__
    if "needs_layout_passes" in fields:
        cp = dataclasses.replace(cp, needs_layout_passes=False)
    if "use_tc_tiling_on_sc" in fields:
        cp = dataclasses.replace(cp, use_tc_tiling_on_sc=False)
    route = pl.kernel(
        body,
        out_type=[
            jax.ShapeDtypeStruct((n, _NB), jnp.float32),
            jax.ShapeDtypeStruct((n, 2), jnp.int32),
        ],
        mesh=plsc.VectorSubcoreMesh(core_axis_name="c", subcore_axis_name="s"),
        scratch_types=[
            pltpu.VMEM((_NB, rpw), jnp.float32),
            pltpu.VMEM((rpw, _NB), jnp.float32),
            pltpu.VMEM((rpw, 2), jnp.int32),
            pltpu.VMEM((_L,), jnp.float32),
        ],
        compiler_params=cp,
    )
    return route(logits_t, inv_temp_vec)


@jax.jit
def kernel(z, W1, b1, W2, b2, temperature):
    inv_temp = 1.0 / (jax.nn.softplus(temperature) + 0.1)
    inv_temp_vec = jnp.full((_L,), inv_temp, jnp.float32)
    logits_t = _tc_logits(z, W1, b1, W2, b2)
    weights, idx = _sc_route(logits_t, inv_temp_vec)
    return weights, idx


# speedup vs baseline: 1.0333x; 1.0052x over previous
import dataclasses

import jax
import jax.numpy as jnp
from jax.experimental import pallas as pl
from jax.experimental.pallas import tpu as pltpu
from jax.experimental.pallas import tpu_sc as plsc

_NB = 8
_BM = 1024
_BN = 512

_CHUNKS = 2

_NC = 2
_NS = 16
_L = 16


def _mlp_block(z_ref, w1_ref, b1_ref, w2_ref, b2_ref, logits_ref, acc_ref):
    j = pl.program_id(1)
    nj = pl.num_programs(1)

    h = jax.lax.dot_general(
        z_ref[...], w1_ref[...], (((1,), (1,)), ((), ())),
        preferred_element_type=jnp.float32)
    h = h + b1_ref[...]
    h = 0.5 * h * (1.0 + jax.lax.erf(h * 0.7071067811865476))
    part = jax.lax.dot_general(
        w2_ref[...], h, (((1,), (1,)), ((), ())),
        preferred_element_type=jnp.float32)

    @pl.when(j == 0)
    def _init():
        acc_ref[...] = part

    @pl.when(j > 0)
    def _accum():
        acc_ref[...] += part

    @pl.when(j == nj - 1)
    def _write():
        logits_ref[...] = acc_ref[...] + b2_ref[...]


def _tc_logits(z, W1, b1, W2, b2, rows, row_off):
    n, d = z.shape
    b1r = jnp.reshape(b1, (1, d))
    b2r = jnp.reshape(b2, (_NB, 1))
    grid = (rows // _BM, d // _BN)
    i_off = row_off // _BM
    return pl.pallas_call(
        _mlp_block,
        grid=grid,
        in_specs=[
            pl.BlockSpec((_BM, d), lambda i, j: (i + i_off, 0)),
            pl.BlockSpec((_BN, d), lambda i, j: (j, 0)),
            pl.BlockSpec((1, _BN), lambda i, j: (0, j)),
            pl.BlockSpec((_NB, _BN), lambda i, j: (0, j)),
            pl.BlockSpec((_NB, 1), lambda i, j: (0, 0)),
        ],
        out_specs=pl.BlockSpec((_NB, _BM), lambda i, j: (0, i)),
        out_shape=jax.ShapeDtypeStruct((_NB, rows), jnp.float32),
        scratch_shapes=[pltpu.VMEM((_NB, _BM), jnp.float32)],
        compiler_params=pltpu.CompilerParams(
            dimension_semantics=("parallel", "arbitrary")),
    )(z, W1, b1r, W2, b2r)


def _sc_route(logits_t, inv_temp_vec):
    n = logits_t.shape[1]
    rpw = n // (_NC * _NS)

    def body(lg_hbm, invt_hbm, w_hbm, idx_hbm, lg_v, w_v, ix_v, invt_v):
        wid = jax.lax.axis_index("s") * _NC + jax.lax.axis_index("c")
        base = wid * rpw
        pltpu.sync_copy(invt_hbm, invt_v)
        pltpu.sync_copy(lg_hbm.at[:, pl.ds(base, rpw)], lg_v)
        inv_t = invt_v[...]
        iota = jax.lax.iota(jnp.int32, _L)

        @pl.loop(0, rpw, step=_L)
        def _(c):
            rows = iota + c
            cols = [jnp.full((_L,), e, jnp.int32) for e in range(_NB)]
            l = [lg_v[e, pl.ds(c, _L)] for e in range(_NB)]
            m1 = l[0]
            for e in range(1, _NB):
                m1 = jnp.maximum(m1, l[e])
            idx1 = jnp.full((_L,), 0, jnp.int32)
            for e in range(_NB - 1, -1, -1):
                idx1 = jnp.where(l[e] == m1, e, idx1)
            neg = jnp.full((_L,), -jnp.inf, jnp.float32)
            m2 = neg
            for e in range(_NB):
                m2 = jnp.maximum(m2, jnp.where(idx1 == e, neg, l[e]))
            idx2 = jnp.full((_L,), 0, jnp.int32)
            for e in range(_NB - 1, -1, -1):
                hit = jnp.logical_and(l[e] == m2, idx1 != e)
                idx2 = jnp.where(hit, e, idx2)
            ex = jnp.exp((m2 - m1) * inv_t)
            denom = 1.0 + ex
            w_hi = 1.0 / denom
            w_lo = ex / denom
            zero = jnp.zeros((_L,), jnp.float32)
            for e in range(_NB):
                we = jnp.where(idx1 == e, w_hi,
                               jnp.where(idx2 == e, w_lo, zero))
                plsc.store_scatter(w_v, [rows, cols[e]], we)
            plsc.store_scatter(ix_v, [rows, cols[0]], idx1)
            plsc.store_scatter(ix_v, [rows, cols[1]], idx2)

        pltpu.sync_copy(w_v, w_hbm.at[pl.ds(base, rpw)])
        pltpu.sync_copy(ix_v, idx_hbm.at[pl.ds(base, rpw)])

    cp = pltpu.CompilerParams()
    fields = pltpu.CompilerParams.__dataclass_fields__
    if "needs_layout_passes" in fields:
        cp = dataclasses.replace(cp, needs_layout_passes=False)
    if "use_tc_tiling_on_sc" in fields:
        cp = dataclasses.replace(cp, use_tc_tiling_on_sc=False)
    route = pl.kernel(
        body,
        out_type=[
            jax.ShapeDtypeStruct((n, _NB), jnp.float32),
            jax.ShapeDtypeStruct((n, 2), jnp.int32),
        ],
        mesh=plsc.VectorSubcoreMesh(core_axis_name="c", subcore_axis_name="s"),
        scratch_types=[
            pltpu.VMEM((_NB, rpw), jnp.float32),
            pltpu.VMEM((rpw, _NB), jnp.float32),
            pltpu.VMEM((rpw, 2), jnp.int32),
            pltpu.VMEM((_L,), jnp.float32),
        ],
        compiler_params=cp,
    )
    return route(logits_t, inv_temp_vec)


@jax.jit
def kernel(z, W1, b1, W2, b2, temperature):
    inv_temp = 1.0 / (jax.nn.softplus(temperature) + 0.1)
    inv_temp_vec = jnp.full((_L,), inv_temp, jnp.float32)
    n = z.shape[0]
    cs = n // _CHUNKS
    parts = []
    for k in range(_CHUNKS):
        logits_t = _tc_logits(z, W1, b1, W2, b2, cs, k * cs)
        parts.append(_sc_route(logits_t, inv_temp_vec))
    if _CHUNKS == 1:
        return parts[0]
    weights = jnp.concatenate([p[0] for p in parts], axis=0)
    idx = jnp.concatenate([p[1] for p in parts], axis=0)
    return weights, idx
